# Initial kernel scaffold; baseline (speedup 1.0000x reference)
import jax
import jax.numpy as jnp
from jax.experimental import pallas as pl


def kernel(codes, tables, channel_emb):
    raise NotImplementedError("write your pallas kernel here")


# speedup vs baseline: 6.9798x; 6.9798x over previous
import functools

import jax
import jax.numpy as jnp
from jax import lax
from jax.experimental import pallas as pl
from jax.experimental.pallas import tpu as pltpu
import jax.experimental.pallas.tpu_sc as plsc

B, T, C, Q, V, D = 1024, 50, 2, 8, 8192, 64
NC, NS = 2, 16
NW = NC * NS
N = B * T * C * Q
RPW = N // NW
CH = 1024
NCHUNK = RPW // CH
GSUB = 128
SUB = CH // GSUB


def _expand_body(tab_ref, bias_ref, out_ref):
    i = pl.program_id(0) * Q + pl.program_id(1)
    out_ref[...] = tab_ref[...] + bias_ref[pl.ds(i, 1), :]


def _expand_table(tables, channel_emb):
    tab2 = tables.reshape(Q * V, D)
    bias = channel_emb.reshape(C * Q, D)
    return pl.pallas_call(
        _expand_body,
        grid=(C, Q),
        in_specs=[
            pl.BlockSpec((V, D), lambda c, q: (q, 0)),
            pl.BlockSpec((C * Q, D), lambda c, q: (0, 0)),
        ],
        out_specs=pl.BlockSpec((V, D), lambda c, q: (c * Q + q, 0)),
        out_shape=jax.ShapeDtypeStruct((C * Q * V, D), jnp.float32),
    )(tab2, bias)


def _gather_body(codes_hbm, exp_hbm, out_hbm, idx_v, rows_v, sem):
    wid = lax.axis_index("s") * NC + lax.axis_index("c")
    base = wid * RPW
    offs = lax.iota(jnp.int32, 16) * V

    @pl.loop(0, NCHUNK)
    def _chunk(g):
        row0 = pl.multiple_of(base + g * CH, CH)
        pltpu.sync_copy(
            codes_hbm.at[pl.ds(pl.multiple_of(row0 // GSUB, SUB), SUB)], idx_v
        )
        for i in range(SUB):
            for j in range(GSUB // 16):
                sl = pl.ds(j * 16, 16)
                idx_v[i, sl] = idx_v[i, sl] + offs
        cps = [
            pltpu.async_copy(
                exp_hbm.at[idx_v.at[i]], rows_v.at[pl.ds(i * GSUB, GSUB)], sem
            )
            for i in range(SUB)
        ]
        for cp in cps:
            cp.wait()
        pltpu.sync_copy(rows_v, out_hbm.at[pl.ds(row0, CH)])


@functools.cache
def _make_gather():
    return pl.kernel(
        _gather_body,
        out_type=jax.ShapeDtypeStruct((N, D), jnp.float32),
        mesh=plsc.VectorSubcoreMesh(
            core_axis_name="c", subcore_axis_name="s", num_cores=NC, num_subcores=NS
        ),
        scratch_types=[
            pltpu.VMEM((SUB, GSUB), jnp.int32),
            pltpu.VMEM((CH, D), jnp.float32),
            pltpu.SemaphoreType.DMA,
        ],
        compiler_params=pltpu.CompilerParams(use_tc_tiling_on_sc=False),
    )


def kernel(codes, tables, channel_emb):
    exp = _expand_table(tables, channel_emb)
    _gather = _make_gather()
    codes2 = codes.astype(jnp.int32).reshape(N // GSUB, GSUB)
    out = _gather(codes2, exp)
    return out.reshape(B, T, C * Q * D)


# speedup vs baseline: 7.0626x; 1.0119x over previous
import functools

import jax
import jax.numpy as jnp
from jax import lax
from jax.experimental import pallas as pl
from jax.experimental.pallas import tpu as pltpu
import jax.experimental.pallas.tpu_sc as plsc

B, T, C, Q, V, D = 1024, 50, 2, 8, 8192, 64
NC, NS = 2, 16
NW = NC * NS
N = B * T * C * Q
RPW = N // NW
CH = 512
NCHUNK = RPW // CH
GSUB = 128
SUB = CH // GSUB


def _expand_body(tab_ref, bias_ref, out_ref):
    i = pl.program_id(0) * Q + pl.program_id(1)
    out_ref[...] = tab_ref[...] + bias_ref[pl.ds(i, 1), :]


def _expand_table(tables, channel_emb):
    tab2 = tables.reshape(Q * V, D)
    bias = channel_emb.reshape(C * Q, D)
    return pl.pallas_call(
        _expand_body,
        grid=(C, Q),
        in_specs=[
            pl.BlockSpec((V, D), lambda c, q: (q, 0)),
            pl.BlockSpec((C * Q, D), lambda c, q: (0, 0)),
        ],
        out_specs=pl.BlockSpec((V, D), lambda c, q: (c * Q + q, 0)),
        out_shape=jax.ShapeDtypeStruct((C * Q * V, D), jnp.float32),
    )(tab2, bias)


def _gather_body(codes_hbm, exp_hbm, out_hbm, idx_a, idx_b, rows_a, rows_b,
                 sem_a, sem_b):
    wid = lax.axis_index("s") * NC + lax.axis_index("c")
    base = wid * RPW
    offs = lax.iota(jnp.int32, 16) * V

    def fire(g, idx_v, rows_v, sem):
        row0 = pl.multiple_of(base + g * CH, CH)
        pltpu.sync_copy(
            codes_hbm.at[pl.ds(pl.multiple_of(row0 // GSUB, SUB), SUB)], idx_v
        )
        for i in range(SUB):
            for j in range(GSUB // 16):
                sl = pl.ds(j * 16, 16)
                idx_v[i, sl] = idx_v[i, sl] + offs
        for i in range(SUB):
            pltpu.async_copy(
                exp_hbm.at[idx_v.at[i]], rows_v.at[pl.ds(i * GSUB, GSUB)], sem
            )

    def drain(idx_v, rows_v, sem):
        for i in range(SUB):
            pltpu.make_async_copy(
                exp_hbm.at[idx_v.at[i]], rows_v.at[pl.ds(i * GSUB, GSUB)], sem
            ).wait()

    def write(g, rows_v):
        row0 = pl.multiple_of(base + g * CH, CH)
        pltpu.sync_copy(rows_v, out_hbm.at[pl.ds(row0, CH)])

    fire(0, idx_a, rows_a, sem_a)

    @pl.loop(0, NCHUNK, step=2)
    def _pair(g):
        fire(g + 1, idx_b, rows_b, sem_b)
        drain(idx_a, rows_a, sem_a)
        write(g, rows_a)
        @pl.when(g + 2 < NCHUNK)
        def _():
            fire(g + 2, idx_a, rows_a, sem_a)

        drain(idx_b, rows_b, sem_b)
        write(g + 1, rows_b)


@functools.cache
def _make_gather():
    return pl.kernel(
        _gather_body,
        out_type=jax.ShapeDtypeStruct((N, D), jnp.float32),
        mesh=plsc.VectorSubcoreMesh(
            core_axis_name="c", subcore_axis_name="s", num_cores=NC, num_subcores=NS
        ),
        scratch_types=[
            pltpu.VMEM((SUB, GSUB), jnp.int32),
            pltpu.VMEM((SUB, GSUB), jnp.int32),
            pltpu.VMEM((CH, D), jnp.float32),
            pltpu.VMEM((CH, D), jnp.float32),
            pltpu.SemaphoreType.DMA,
            pltpu.SemaphoreType.DMA,
        ],
        compiler_params=pltpu.CompilerParams(use_tc_tiling_on_sc=False),
    )


def kernel(codes, tables, channel_emb):
    exp = _expand_table(tables, channel_emb)
    _gather = _make_gather()
    codes2 = codes.astype(jnp.int32).reshape(N // GSUB, GSUB)
    out = _gather(codes2, exp)
    return out.reshape(B, T, C * Q * D)


# speedup vs baseline: 7.5378x; 1.0673x over previous
import functools

import jax
import jax.numpy as jnp
from jax import lax
from jax.experimental import pallas as pl
from jax.experimental.pallas import tpu as pltpu
import jax.experimental.pallas.tpu_sc as plsc

B, T, C, Q, V, D = 1024, 50, 2, 8, 8192, 64
NC, NS = 2, 16
NW = NC * NS
N = B * T * C * Q
RPW = N // NW
CH = 512
NCHUNK = RPW // CH
GSUB = 128
SUB = CH // GSUB


def _expand_body(tab_ref, ch_ref, out_ref):
    q = pl.program_id(0)
    t = tab_ref[0]
    b0 = ch_ref[pl.ds(q, 1), :]
    b1 = ch_ref[pl.ds(Q + q, 1), :]
    out_ref[...] = jnp.concatenate([t + b0, t + b1], axis=1)


def _expand_table(tables, channel_emb):
    return pl.pallas_call(
        _expand_body,
        grid=(Q,),
        in_specs=[
            pl.BlockSpec((1, V, D), lambda q: (q, 0, 0)),
            pl.BlockSpec((C * Q, D), lambda q: (0, 0)),
        ],
        out_specs=pl.BlockSpec((V, 2 * D), lambda q: (q, 0)),
        out_shape=jax.ShapeDtypeStruct((Q * V, 2 * D), jnp.float32),
    )(tables, channel_emb.reshape(C * Q, D))


def _gather_body(codes_hbm, exp_hbm, out_hbm, idx_a, idx_b, rows_a, rows_b,
                 sem_a, sem_b):
    wid = lax.axis_index("s") * NC + lax.axis_index("c")
    base = wid * RPW
    lane = lax.iota(jnp.int32, 16)
    offs = ((lane & (Q - 1)) << 14) + (lane >> 3)

    def fire(g, idx_v, rows_v, sem):
        row0 = pl.multiple_of(base + g * CH, CH)
        pltpu.sync_copy(
            codes_hbm.at[pl.ds(pl.multiple_of(row0 // GSUB, SUB), SUB)], idx_v
        )
        for i in range(SUB):
            for j in range(GSUB // 16):
                sl = pl.ds(j * 16, 16)
                idx_v[i, sl] = idx_v[i, sl] * 2 + offs
        for i in range(SUB):
            pltpu.async_copy(
                exp_hbm.at[idx_v.at[i]], rows_v.at[pl.ds(i * GSUB, GSUB)], sem
            )

    def drain(idx_v, rows_v, sem):
        for i in range(SUB):
            pltpu.make_async_copy(
                exp_hbm.at[idx_v.at[i]], rows_v.at[pl.ds(i * GSUB, GSUB)], sem
            ).wait()

    def write(g, rows_v):
        row0 = pl.multiple_of(base + g * CH, CH)
        pltpu.sync_copy(rows_v, out_hbm.at[pl.ds(row0, CH)])

    fire(0, idx_a, rows_a, sem_a)

    @pl.loop(0, NCHUNK, step=2)
    def _pair(g):
        fire(g + 1, idx_b, rows_b, sem_b)
        drain(idx_a, rows_a, sem_a)
        write(g, rows_a)
        @pl.when(g + 2 < NCHUNK)
        def _():
            fire(g + 2, idx_a, rows_a, sem_a)

        drain(idx_b, rows_b, sem_b)
        write(g + 1, rows_b)


@functools.cache
def _make_gather():
    return pl.kernel(
        _gather_body,
        out_type=jax.ShapeDtypeStruct((N, D), jnp.float32),
        mesh=plsc.VectorSubcoreMesh(
            core_axis_name="c", subcore_axis_name="s", num_cores=NC, num_subcores=NS
        ),
        scratch_types=[
            pltpu.VMEM((SUB, GSUB), jnp.int32),
            pltpu.VMEM((SUB, GSUB), jnp.int32),
            pltpu.VMEM((CH, D), jnp.float32),
            pltpu.VMEM((CH, D), jnp.float32),
            pltpu.SemaphoreType.DMA,
            pltpu.SemaphoreType.DMA,
        ],
        compiler_params=pltpu.CompilerParams(use_tc_tiling_on_sc=False),
    )


def kernel(codes, tables, channel_emb):
    exp = _expand_table(tables, channel_emb).reshape(C * Q * V, D)
    _gather = _make_gather()
    codes2 = codes.astype(jnp.int32).reshape(N // GSUB, GSUB)
    out = _gather(codes2, exp)
    return out.reshape(B, T, C * Q * D)
